# baseline (speedup 1.0000x reference)
import functools

import jax
import jax.numpy as jnp
from jax.experimental import pallas as pl
from jax.experimental.pallas import tpu as pltpu


def _hgc_kernel(h_ref, x_ref, out_ref, rs_ref, *, nsteps, tag_num, tt):
    k = pl.program_id(0)
    lane = jax.lax.broadcasted_iota(jnp.int32, (1, tt), 1)
    valid = (lane < (tag_num - k * tt)).astype(jnp.float32)
    h = (h_ref[...] > 0).astype(jnp.float32) * valid
    x = x_ref[...]

    col = jnp.sum(h, axis=0)
    b_inv = 1.0 / jnp.where(col == 0.0, 1.0, col)

    m = jax.lax.dot_general(
        h, x, (((0,), (0,)), ((), ())), preferred_element_type=jnp.float32
    )
    m = m * b_inv[:, None]

    contrib = jnp.dot(h, m, preferred_element_type=jnp.float32)

    rs = jnp.sum(h, axis=1, keepdims=True)

    @pl.when(k == 0)
    def _init():
        out_ref[...] = contrib
        rs_ref[...] = rs

    @pl.when(k != 0)
    def _acc():
        out_ref[...] += contrib
        rs_ref[...] += rs

    @pl.when(k == nsteps - 1)
    def _finish():
        d = rs_ref[...]
        d = jnp.where(d == 0.0, 1.0, d)
        out_ref[...] = out_ref[...] / d


@jax.jit
def kernel(item_embeds, H):
    item_num, dim = item_embeds.shape
    tag_num = H.shape[1]
    tt = 128
    nsteps = pl.cdiv(tag_num, tt)

    return pl.pallas_call(
        functools.partial(_hgc_kernel, nsteps=nsteps, tag_num=tag_num, tt=tt),
        grid=(nsteps,),
        in_specs=[
            pl.BlockSpec((item_num, tt), lambda k: (0, k)),
            pl.BlockSpec((item_num, dim), lambda k: (0, 0)),
        ],
        out_specs=pl.BlockSpec((item_num, dim), lambda k: (0, 0)),
        out_shape=jax.ShapeDtypeStruct((item_num, dim), jnp.float32),
        scratch_shapes=[pltpu.VMEM((item_num, 1), jnp.float32)],
        compiler_params=pltpu.CompilerParams(
            dimension_semantics=("arbitrary",),
        ),
    )(H, item_embeds)
